# Initial kernel scaffold; baseline (speedup 1.0000x reference)
import jax
import jax.numpy as jnp
from jax.experimental import pallas as pl


def kernel(batch_box_preds, batch_cls_preds):
    raise NotImplementedError("write your pallas kernel here")


# speedup vs baseline: 17.5528x; 17.5528x over previous
import jax
import jax.numpy as jnp
from jax.experimental import pallas as pl
from jax.experimental.pallas import tpu as pltpu

_B = 4
_N = 20000
_NUM_CLS = 3
_PRE = 2048
_POST = 500
_THRESH = 0.7
_SLOTS = 512
_TILE = 256
_F = 16


def _nms_body(feat_ref, featT_ref, out_ref, adj_ref, tri_ref):
    xr = featT_ref[0, 0:1, :]
    yr = featT_ref[0, 1:2, :]
    dxr = featT_ref[0, 3:4, :]
    dyr = featT_ref[0, 4:5, :]
    x1r = xr - dxr * 0.5
    x2r = xr + dxr * 0.5
    y1r = yr - dyr * 0.5
    y2r = yr + dyr * 0.5
    ar = dxr * dyr

    def build(t, carry):
        ft = feat_ref[0, pl.ds(t * _TILE, _TILE), :]
        xc = ft[:, 0:1]
        yc = ft[:, 1:2]
        dxc = ft[:, 3:4]
        dyc = ft[:, 4:5]
        x1c = xc - dxc * 0.5
        x2c = xc + dxc * 0.5
        y1c = yc - dyc * 0.5
        y2c = yc + dyc * 0.5
        ac = dxc * dyc
        ix = jnp.clip(jnp.minimum(x2c, x2r) - jnp.maximum(x1c, x1r), 0.0)
        iy = jnp.clip(jnp.minimum(y2c, y2r) - jnp.maximum(y1c, y1r), 0.0)
        inter = ix * iy
        union = ac + ar - inter
        iou = inter / jnp.maximum(union, 1e-6)
        rowi = jax.lax.broadcasted_iota(jnp.int32, (_TILE, _PRE), 0) + t * _TILE
        coli = jax.lax.broadcasted_iota(jnp.int32, (_TILE, _PRE), 1)
        upper = coli > rowi
        adj_ref[pl.ds(t * _TILE, _TILE), :] = jnp.where(
            (iou > _THRESH) & upper, 1.0, 0.0).astype(jnp.bfloat16)
        tri_ref[pl.ds(t * _TILE, _TILE), :] = jnp.where(
            upper, 1.0, 0.0).astype(jnp.bfloat16)
        return carry

    jax.lax.fori_loop(0, _PRE // _TILE, build, 0)

    def cond(c):
        return c[1]

    def body(c):
        keep, _ = c
        supp = jax.lax.dot_general(
            keep, adj_ref[...], (((1,), (0,)), ((), ())),
            preferred_element_type=jnp.float32)
        new = (supp == 0.0).astype(jnp.bfloat16)
        changed = jnp.sum(jnp.abs(new.astype(jnp.float32) - keep.astype(jnp.float32))) > 0.0
        return new, changed

    keep0 = jnp.ones((1, _PRE), jnp.bfloat16)
    keep, _ = jax.lax.while_loop(cond, body, (keep0, jnp.bool_(True)))

    p = jax.lax.dot_general(
        keep, tri_ref[...], (((1,), (0,)), ((), ())),
        preferred_element_type=jnp.float32)
    keep32 = keep.astype(jnp.float32)

    srow = jax.lax.broadcasted_iota(jnp.int32, (_SLOTS, _PRE), 0).astype(jnp.float32)
    sel = jnp.where((p == srow) & (keep32 > 0.5), 1.0, 0.0)
    out_ref[0] = jax.lax.dot_general(
        sel, feat_ref[0], (((1,), (0,)), ((), ())),
        precision=jax.lax.Precision.HIGHEST,
        preferred_element_type=jnp.float32)


def kernel(batch_box_preds, batch_cls_preds):
    scores = jnp.max(batch_cls_preds, axis=-1)
    labels = jnp.argmax(batch_cls_preds, axis=-1)
    top_scores, top_idx = jax.lax.top_k(scores, _PRE)
    b = jnp.take_along_axis(batch_box_preds, top_idx[..., None], axis=1)
    l = jnp.take_along_axis(labels, top_idx, axis=1)

    feat = jnp.concatenate(
        [b, top_scores[..., None], (l + 1).astype(jnp.float32)[..., None],
         jnp.zeros((_B, _PRE, _F - 9), jnp.float32)], axis=-1)
    featT = jnp.transpose(feat, (0, 2, 1))

    out = pl.pallas_call(
        _nms_body,
        grid=(_B,),
        in_specs=[
            pl.BlockSpec((1, _PRE, _F), lambda i: (i, 0, 0)),
            pl.BlockSpec((1, _F, _PRE), lambda i: (i, 0, 0)),
        ],
        out_specs=pl.BlockSpec((1, _SLOTS, _F), lambda i: (i, 0, 0)),
        out_shape=jax.ShapeDtypeStruct((_B, _SLOTS, _F), jnp.float32),
        scratch_shapes=[
            pltpu.VMEM((_PRE, _PRE), jnp.bfloat16),
            pltpu.VMEM((_PRE, _PRE), jnp.bfloat16),
        ],
    )(feat, featT)

    rois = out[:, :_POST, 0:7]
    roi_scores = out[:, :_POST, 7]
    roi_labels = jnp.round(out[:, :_POST, 8]).astype(jnp.int32)
    return rois, roi_scores, roi_labels


# speedup vs baseline: 17.5662x; 1.0008x over previous
import jax
import jax.numpy as jnp
from jax.experimental import pallas as pl
from jax.experimental.pallas import tpu as pltpu

_B = 4
_N = 20000
_NUM_CLS = 3
_PRE = 2048
_POST = 500
_THRESH = 0.7
_SLOTS = 512
_TILE = 256
_F = 16


def _nms_body(feat_ref, featT_ref, out_ref, adj_ref, tri_ref):
    xr = featT_ref[0, 0:1, :]
    yr = featT_ref[0, 1:2, :]
    dxr = featT_ref[0, 3:4, :]
    dyr = featT_ref[0, 4:5, :]
    x1r = xr - dxr * 0.5
    x2r = xr + dxr * 0.5
    y1r = yr - dyr * 0.5
    y2r = yr + dyr * 0.5
    ar = dxr * dyr

    def build(t, carry):
        ft = feat_ref[0, pl.ds(t * _TILE, _TILE), :]
        xc = ft[:, 0:1]
        yc = ft[:, 1:2]
        dxc = ft[:, 3:4]
        dyc = ft[:, 4:5]
        x1c = xc - dxc * 0.5
        x2c = xc + dxc * 0.5
        y1c = yc - dyc * 0.5
        y2c = yc + dyc * 0.5
        ac = dxc * dyc
        ix = jnp.clip(jnp.minimum(x2c, x2r) - jnp.maximum(x1c, x1r), 0.0)
        iy = jnp.clip(jnp.minimum(y2c, y2r) - jnp.maximum(y1c, y1r), 0.0)
        inter = ix * iy
        union = ac + ar - inter
        iou = inter / jnp.maximum(union, 1e-6)
        rowi = jax.lax.broadcasted_iota(jnp.int32, (_TILE, _PRE), 0) + t * _TILE
        coli = jax.lax.broadcasted_iota(jnp.int32, (_TILE, _PRE), 1)
        upper = coli > rowi
        adj_ref[pl.ds(t * _TILE, _TILE), :] = jnp.where(
            (iou > _THRESH) & upper, 1.0, 0.0).astype(jnp.bfloat16)
        tri_ref[pl.ds(t * _TILE, _TILE), :] = jnp.where(
            upper, 1.0, 0.0).astype(jnp.bfloat16)
        return carry

    jax.lax.fori_loop(0, _PRE // _TILE, build, 0)

    def cond(c):
        return c[1]

    def body(c):
        keep, _ = c
        supp = jax.lax.dot_general(
            keep, adj_ref[...], (((1,), (0,)), ((), ())),
            preferred_element_type=jnp.float32)
        new = (supp == 0.0).astype(jnp.bfloat16)
        changed = jnp.sum(jnp.abs(new.astype(jnp.float32) - keep.astype(jnp.float32))) > 0.0
        return new, changed

    keep0 = jnp.ones((1, _PRE), jnp.bfloat16)
    keep, _ = jax.lax.while_loop(cond, body, (keep0, jnp.bool_(True)))

    p = jax.lax.dot_general(
        keep, tri_ref[...], (((1,), (0,)), ((), ())),
        preferred_element_type=jnp.float32)
    keep32 = keep.astype(jnp.float32)

    srow = jax.lax.broadcasted_iota(jnp.int32, (_SLOTS, _PRE), 0).astype(jnp.float32)
    sel = jnp.where((p == srow) & (keep32 > 0.5), 1.0, 0.0)
    out_ref[0] = jax.lax.dot_general(
        sel, feat_ref[0], (((1,), (0,)), ((), ())),
        precision=jax.lax.Precision.HIGHEST,
        preferred_element_type=jnp.float32)


def kernel(batch_box_preds, batch_cls_preds):
    scores = jnp.max(batch_cls_preds, axis=-1)
    labels = jnp.argmax(batch_cls_preds, axis=-1)
    top_scores, top_idx = jax.lax.top_k(scores, _PRE)
    b = jnp.take_along_axis(batch_box_preds, top_idx[..., None], axis=1)
    l = jnp.take_along_axis(labels, top_idx, axis=1)

    feat = jnp.concatenate(
        [b, top_scores[..., None], (l + 1).astype(jnp.float32)[..., None],
         jnp.zeros((_B, _PRE, _F - 9), jnp.float32)], axis=-1)
    featT = jnp.transpose(feat, (0, 2, 1))

    out = pl.pallas_call(
        _nms_body,
        grid=(_B,),
        in_specs=[
            pl.BlockSpec((1, _PRE, _F), lambda i: (i, 0, 0)),
            pl.BlockSpec((1, _F, _PRE), lambda i: (i, 0, 0)),
        ],
        out_specs=pl.BlockSpec((1, _SLOTS, _F), lambda i: (i, 0, 0)),
        out_shape=jax.ShapeDtypeStruct((_B, _SLOTS, _F), jnp.float32),
        compiler_params=pltpu.CompilerParams(
            dimension_semantics=("parallel",)),
        scratch_shapes=[
            pltpu.VMEM((_PRE, _PRE), jnp.bfloat16),
            pltpu.VMEM((_PRE, _PRE), jnp.bfloat16),
        ],
    )(feat, featT)

    rois = out[:, :_POST, 0:7]
    roi_scores = out[:, :_POST, 7]
    roi_labels = jnp.round(out[:, :_POST, 8]).astype(jnp.int32)
    return rois, roi_scores, roi_labels
